# Initial kernel scaffold; baseline (speedup 1.0000x reference)
import jax
import jax.numpy as jnp
from jax.experimental import pallas as pl


def kernel(node_features, edge_index, text_embeddings, edge_weights, params):
    raise NotImplementedError("write your pallas kernel here")


# speedup vs baseline: 14.5372x; 14.5372x over previous
import functools
import math

import numpy as np
import jax
import jax.numpy as jnp
from jax import lax
from jax.experimental import pallas as pl
from jax.experimental.pallas import tpu as pltpu
from jax.experimental.pallas import tpu_sc as plsc

H = 8
DH = 32
D_HID = 256
_SCALE = 1.0 / math.sqrt(DH)

_NW = 32
_GC = 80


def _mm_body(x_ref, w_ref, b_ref, o_ref):
    o_ref[...] = (jnp.dot(x_ref[...], w_ref[...],
                          preferred_element_type=jnp.float32) + b_ref[...])


def _mm(x, W, b, bm):
    M, K = x.shape
    D = W.shape[1]
    return pl.pallas_call(
        _mm_body,
        grid=(M // bm,),
        in_specs=[pl.BlockSpec((bm, K), lambda i: (i, 0)),
                  pl.BlockSpec((K, D), lambda i: (0, 0)),
                  pl.BlockSpec((1, D), lambda i: (0, 0))],
        out_specs=pl.BlockSpec((bm, D), lambda i: (i, 0)),
        out_shape=jax.ShapeDtypeStruct((M, D), jnp.float32),
    )(x, W, b.reshape(1, D))


def _scores_body(q_ref, k_ref, ew_ref, smat_ref, webe_ref, o_ref, m_ref):
    i = pl.program_id(0)
    s = jnp.dot(q_ref[...] * k_ref[...], smat_ref[...],
                preferred_element_type=jnp.float32) * _SCALE
    s = s + ew_ref[...] * webe_ref[0:1, :] + webe_ref[1:2, :]
    o_ref[...] = s

    @pl.when(i == 0)
    def _():
        m_ref[...] = jnp.full(m_ref.shape, -jnp.inf, jnp.float32)

    m_ref[...] = jnp.maximum(m_ref[...], jnp.max(s))


def _scores(qg, kg, ew2, smat, webe, be):
    E = qg.shape[0]
    return pl.pallas_call(
        _scores_body,
        grid=(E // be,),
        in_specs=[pl.BlockSpec((be, D_HID), lambda i: (i, 0)),
                  pl.BlockSpec((be, D_HID), lambda i: (i, 0)),
                  pl.BlockSpec((be, 1), lambda i: (i, 0)),
                  pl.BlockSpec((D_HID, H), lambda i: (0, 0)),
                  pl.BlockSpec((2, H), lambda i: (0, 0))],
        out_specs=[pl.BlockSpec((be, H), lambda i: (i, 0)),
                   pl.BlockSpec((1, 128), lambda i: (0, 0))],
        out_shape=[jax.ShapeDtypeStruct((E, H), jnp.float32),
                   jax.ShapeDtypeStruct((1, 128), jnp.float32)],
    )(qg, kg, ew2, smat, webe)


def _weight_body(s_ref, m_ref, v_ref, stmat_ref, wv_ref):
    ex = jnp.exp(s_ref[...] - m_ref[0:1, 0:1])
    exx = jnp.dot(ex, stmat_ref[...], preferred_element_type=jnp.float32)
    w = exx * v_ref[...]
    be = w.shape[0]
    z = jnp.zeros((be, 14), jnp.float32)
    for g in range(4):
        wv_ref[g, :, :] = jnp.concatenate(
            [w[:, g * 64:(g + 1) * 64], ex[:, 2 * g:2 * g + 2], z], axis=-1)


def _weight(scores, m, vg, stmat, be):
    E = scores.shape[0]
    return pl.pallas_call(
        _weight_body,
        grid=(E // be,),
        in_specs=[pl.BlockSpec((be, H), lambda i: (i, 0)),
                  pl.BlockSpec((1, 128), lambda i: (0, 0)),
                  pl.BlockSpec((be, D_HID), lambda i: (i, 0)),
                  pl.BlockSpec((H, D_HID), lambda i: (0, 0))],
        out_specs=pl.BlockSpec((4, be, 80), lambda i: (0, i, 0)),
        out_shape=jax.ShapeDtypeStruct((4, E, 80), jnp.float32),
    )(scores, m, vg, stmat)


def _post_body(agg_ref, h_ref, wo_ref, bo_ref, g_ref, bb_ref,
               s2_ref, o_ref):
    parts = []
    for g in range(4):
        d = jnp.dot(agg_ref[g, :, 64:66], s2_ref[...],
                    preferred_element_type=jnp.float32) + 1e-8
        parts.append(agg_ref[g, :, 0:64] / d)
    att = jnp.concatenate(parts, axis=-1)
    y = (jnp.dot(att, wo_ref[...], preferred_element_type=jnp.float32)
         + bo_ref[...] + h_ref[...])
    mu = jnp.mean(y, axis=-1, keepdims=True)
    var = jnp.mean((y - mu) ** 2, axis=-1, keepdims=True)
    o_ref[...] = (y - mu) * lax.rsqrt(var + 1e-5) * g_ref[...] + bb_ref[...]


def _post(agg, h, Wo, bo, g, b, s2, bn):
    N = h.shape[0]
    return pl.pallas_call(
        _post_body,
        grid=(N // bn,),
        in_specs=[pl.BlockSpec((4, bn, 80), lambda i: (0, i, 0)),
                  pl.BlockSpec((bn, D_HID), lambda i: (i, 0)),
                  pl.BlockSpec((D_HID, D_HID), lambda i: (0, 0)),
                  pl.BlockSpec((1, D_HID), lambda i: (0, 0)),
                  pl.BlockSpec((1, D_HID), lambda i: (0, 0)),
                  pl.BlockSpec((1, D_HID), lambda i: (0, 0)),
                  pl.BlockSpec((2, 64), lambda i: (0, 0))],
        out_specs=pl.BlockSpec((bn, D_HID), lambda i: (i, 0)),
        out_shape=jax.ShapeDtypeStruct((N, D_HID), jnp.float32),
    )(agg, h, Wo, bo.reshape(1, -1), g.reshape(1, -1),
      b.reshape(1, -1), s2)


def _crossprep_body(t_ref, wt_ref, bt_ref, wk_ref, bk_ref, wv_ref, bv_ref,
                    k_ref, v_ref):
    tp = (jnp.dot(t_ref[...], wt_ref[...],
                  preferred_element_type=jnp.float32) + bt_ref[...])
    k_ref[...] = (jnp.dot(tp, wk_ref[...],
                          preferred_element_type=jnp.float32) + bk_ref[...])
    v_ref[...] = (jnp.dot(tp, wv_ref[...],
                          preferred_element_type=jnp.float32) + bv_ref[...])


def _crossprep(text_pad, cp):
    TP, DT = text_pad.shape
    return pl.pallas_call(
        _crossprep_body,
        grid=(1,),
        in_specs=[pl.BlockSpec((TP, DT), lambda i: (0, 0)),
                  pl.BlockSpec((DT, D_HID), lambda i: (0, 0)),
                  pl.BlockSpec((1, D_HID), lambda i: (0, 0)),
                  pl.BlockSpec((D_HID, D_HID), lambda i: (0, 0)),
                  pl.BlockSpec((1, D_HID), lambda i: (0, 0)),
                  pl.BlockSpec((D_HID, D_HID), lambda i: (0, 0)),
                  pl.BlockSpec((1, D_HID), lambda i: (0, 0))],
        out_specs=[pl.BlockSpec((TP, D_HID), lambda i: (0, 0)),
                   pl.BlockSpec((TP, D_HID), lambda i: (0, 0))],
        out_shape=[jax.ShapeDtypeStruct((TP, D_HID), jnp.float32),
                   jax.ShapeDtypeStruct((TP, D_HID), jnp.float32)],
    )(text_pad, cp["t"]["W"], cp["t"]["b"].reshape(1, -1),
      cp["k"]["W"], cp["k"]["b"].reshape(1, -1),
      cp["v"]["W"], cp["v"]["b"].reshape(1, -1))


def _crossmain_body(h_ref, k_ref, v_ref, wr_ref, br_ref, wq_ref, bq_ref,
                    wo_ref, bo_ref, g_ref, bb_ref, o_ref, *, t_real):
    rp = (jnp.dot(h_ref[...], wr_ref[...],
                  preferred_element_type=jnp.float32) + br_ref[...])
    q = (jnp.dot(rp, wq_ref[...],
                 preferred_element_type=jnp.float32) + bq_ref[...])
    kk = k_ref[...]
    vv = v_ref[...]
    tp = kk.shape[0]
    mask = (lax.broadcasted_iota(jnp.int32, (1, tp), 1) < t_real)
    outs = []
    for hh in range(H):
        qh = q[:, hh * DH:(hh + 1) * DH]
        kh = kk[:, hh * DH:(hh + 1) * DH]
        vh = vv[:, hh * DH:(hh + 1) * DH]
        sc = jax.lax.dot_general(
            qh, kh, (((1,), (1,)), ((), ())),
            preferred_element_type=jnp.float32) * _SCALE
        sc = jnp.where(mask, sc, -1e30)
        sc = sc - jnp.max(sc, axis=-1, keepdims=True)
        e = jnp.exp(sc)
        p = e / jnp.sum(e, axis=-1, keepdims=True)
        outs.append(jnp.dot(p, vh, preferred_element_type=jnp.float32))
    att = jnp.concatenate(outs, axis=-1)
    y = (jnp.dot(att, wo_ref[...], preferred_element_type=jnp.float32)
         + bo_ref[...] + h_ref[...])
    mu = jnp.mean(y, axis=-1, keepdims=True)
    var = jnp.mean((y - mu) ** 2, axis=-1, keepdims=True)
    o_ref[...] = (y - mu) * lax.rsqrt(var + 1e-5) * g_ref[...] + bb_ref[...]


def _crossmain(h, K, V, cp, t_real, bn):
    N = h.shape[0]
    TP = K.shape[0]
    body = functools.partial(_crossmain_body, t_real=t_real)
    return pl.pallas_call(
        body,
        grid=(N // bn,),
        in_specs=[pl.BlockSpec((bn, D_HID), lambda i: (i, 0)),
                  pl.BlockSpec((TP, D_HID), lambda i: (0, 0)),
                  pl.BlockSpec((TP, D_HID), lambda i: (0, 0)),
                  pl.BlockSpec((D_HID, D_HID), lambda i: (0, 0)),
                  pl.BlockSpec((1, D_HID), lambda i: (0, 0)),
                  pl.BlockSpec((D_HID, D_HID), lambda i: (0, 0)),
                  pl.BlockSpec((1, D_HID), lambda i: (0, 0)),
                  pl.BlockSpec((D_HID, D_HID), lambda i: (0, 0)),
                  pl.BlockSpec((1, D_HID), lambda i: (0, 0)),
                  pl.BlockSpec((1, D_HID), lambda i: (0, 0)),
                  pl.BlockSpec((1, D_HID), lambda i: (0, 0))],
        out_specs=pl.BlockSpec((bn, D_HID), lambda i: (i, 0)),
        out_shape=jax.ShapeDtypeStruct((N, D_HID), jnp.float32),
    )(h, K, V, cp["r"]["W"], cp["r"]["b"].reshape(1, -1),
      cp["q"]["W"], cp["q"]["b"].reshape(1, -1),
      cp["o"]["W"], cp["o"]["b"].reshape(1, -1),
      cp["ln_g"].reshape(1, -1), cp["ln_b"].reshape(1, -1))


def _sc_gather3(Q, K, V, tgt, src):
    N = Q.shape[0]
    E = tgt.shape[0]
    per_w = E // _NW
    mesh = plsc.VectorSubcoreMesh(core_axis_name="c", subcore_axis_name="s")

    @functools.partial(
        pl.kernel, mesh=mesh,
        out_type=[jax.ShapeDtypeStruct((E, D_HID), jnp.float32),
                  jax.ShapeDtypeStruct((E, D_HID), jnp.float32),
                  jax.ShapeDtypeStruct((E, D_HID), jnp.float32)],
        scratch_types=[pltpu.VMEM((_GC,), jnp.int32),
                       pltpu.VMEM((_GC,), jnp.int32),
                       pltpu.VMEM((_GC, D_HID), jnp.float32),
                       pltpu.SemaphoreType.DMA],
    )
    def k(q_hbm, k_hbm, v_hbm, tgt_hbm, src_hbm, qg_hbm, kg_hbm, vg_hbm,
          tidx, sidx, rows, sem):
        wid = lax.axis_index("s") * 2 + lax.axis_index("c")
        base = wid * per_w

        @pl.loop(0, per_w, step=_GC)
        def _(i):
            b = base + i
            pltpu.sync_copy(tgt_hbm.at[pl.ds(b, _GC)], tidx)
            pltpu.sync_copy(src_hbm.at[pl.ds(b, _GC)], sidx)
            pltpu.async_copy(q_hbm.at[tidx], rows, sem).wait()
            pltpu.sync_copy(rows, qg_hbm.at[pl.ds(b, _GC)])
            pltpu.async_copy(k_hbm.at[sidx], rows, sem).wait()
            pltpu.sync_copy(rows, kg_hbm.at[pl.ds(b, _GC)])
            pltpu.async_copy(v_hbm.at[sidx], rows, sem).wait()
            pltpu.sync_copy(rows, vg_hbm.at[pl.ds(b, _GC)])

    return k(Q, K, V, tgt, src)


def _sc_scatter(tgt, wv, N):
    E = tgt.shape[0]
    per_t = E // 16
    n_writers = 10
    n_per_t = N // n_writers
    zrows = 200
    mesh = plsc.VectorSubcoreMesh(core_axis_name="c", subcore_axis_name="s")

    @functools.partial(
        pl.kernel, mesh=mesh,
        out_type=jax.ShapeDtypeStruct((4, N, 80), jnp.float32),
        scratch_types=[pltpu.VMEM((_GC,), jnp.int32),
                       pltpu.VMEM((_GC, 80), jnp.float32),
                       pltpu.VMEM((zrows, 80), jnp.float32),
                       pltpu.VMEM_SHARED((N, 80), jnp.float32),
                       pltpu.SemaphoreType.DMA],
    )
    def k(tgt_hbm, wv_hbm, agg_hbm, tidx, wrows, zb, acc, sem):
        c = lax.axis_index("c")
        t = lax.axis_index("s")

        @pl.loop(0, zrows)
        def _(r):
            for j in range(5):
                zb[r, pl.ds(j * 16, 16)] = jnp.zeros((16,), jnp.float32)

        base = t * per_t
        for gi in range(2):
            g = 2 * c + gi

            @pl.when(t < n_writers)
            def _():
                @pl.loop(0, n_per_t, step=zrows)
                def _(i):
                    pltpu.sync_copy(zb, acc.at[pl.ds(t * n_per_t + i, zrows)])

            plsc.subcore_barrier()

            @pl.loop(0, per_t, step=_GC)
            def _(i):
                b = base + i
                pltpu.sync_copy(tgt_hbm.at[pl.ds(b, _GC)], tidx)
                pltpu.sync_copy(wv_hbm.at[g, pl.ds(b, _GC)], wrows)
                pltpu.sync_copy(wrows, acc.at[tidx], add=True)

            plsc.subcore_barrier()

            @pl.when(t < n_writers)
            def _():
                r0 = t * n_per_t
                pltpu.sync_copy(acc.at[pl.ds(r0, n_per_t)],
                                agg_hbm.at[g, pl.ds(r0, n_per_t)])

            plsc.subcore_barrier()

    return k(tgt, wv)


def _head_sum_mat():
    return jnp.asarray(
        (np.arange(D_HID)[:, None] // DH) == np.arange(H)[None, :],
        jnp.float32)


def kernel(node_features, edge_index, text_embeddings, edge_weights, params):
    N = node_features.shape[0]
    src = edge_index[0]
    tgt = edge_index[1]
    ew2 = edge_weights.reshape(-1, 1)

    smat = _head_sum_mat()
    stmat = smat.T
    s2 = jnp.asarray(
        (np.arange(64)[:, None] // DH) == np.arange(2)[None, :],
        jnp.float32).T

    h = _mm(node_features, params["in_proj"]["W"], params["in_proj"]["b"],
            1000)

    for lp in params["layers"]:
        Q = _mm(h, lp["q"]["W"], lp["q"]["b"], 1000)
        K = _mm(h, lp["k"]["W"], lp["k"]["b"], 1000)
        V = _mm(h, lp["v"]["W"], lp["v"]["b"], 1000)
        qg, kg, vg = _sc_gather3(Q, K, V, tgt, src)
        webe = jnp.stack([lp["e"]["W"].reshape(H), lp["e"]["b"]])
        scores, m = _scores(qg, kg, ew2, smat, webe, 1000)
        wv = _weight(scores, m, vg, stmat, 1000)
        agg = _sc_scatter(tgt, wv, N)
        h = _post(agg, h, lp["o"]["W"], lp["o"]["b"],
                  lp["ln_g"], lp["ln_b"], s2, 1000)

    cp = params["cross"]
    t_real = text_embeddings.shape[0]
    tp_pad = 8 * ((t_real + 7) // 8)
    text_pad = jnp.pad(text_embeddings, ((0, tp_pad - t_real), (0, 0)))
    Kc, Vc = _crossprep(text_pad, cp)
    out = _crossmain(h, Kc, Vc, cp, t_real, 1000)
    return out
